# Initial kernel scaffold; baseline (speedup 1.0000x reference)
import jax
import jax.numpy as jnp
from jax.experimental import pallas as pl


def kernel(x, atom_table, graph_token):
    raise NotImplementedError("write your pallas kernel here")


# speedup vs baseline: 5.5818x; 5.5818x over previous
import numpy as np
import jax
import jax.numpy as jnp
from jax import lax
from jax.experimental import pallas as pl
from jax.experimental.pallas import tpu as pltpu
from jax.experimental.pallas import tpu_sc as plsc

B, N, F, D = 1024, 64, 8, 128
IDX_PER_BATCH = N * F
IDX_ROWS = IDX_PER_BATCH // 128
NW = 32
BATCHES_PER_W = B // NW
OUT_ROWS = N + 1
SLOT = 72


def _sc_body(x_hbm, tab_hbm, tmpl_hbm, tgt_hbm, out_hbm,
             idx_v, tgt_v, rows_v, tmpl_v, acc_sh):
    c = lax.axis_index("c")
    s = lax.axis_index("s")
    wid = s * 2 + c
    first = wid * BATCHES_PER_W

    pltpu.sync_copy(tmpl_hbm, tmpl_v)
    pltpu.sync_copy(tgt_hbm.at[s], tgt_v)
    base0 = s * 2 * SLOT

    @pl.loop(0, BATCHES_PER_W, step=2)
    def _(t):
        for u in range(2):
            b = first + t + u
            slot = base0 + u * SLOT
            pltpu.sync_copy(x_hbm.at[b], idx_v)
            pltpu.sync_copy(tmpl_v, acc_sh.at[pl.ds(slot, OUT_ROWS)])
            for j in range(IDX_ROWS):
                pltpu.sync_copy(tab_hbm.at[idx_v.at[j]],
                                rows_v.at[pl.ds(j * 128, 128)])
            for j in range(IDX_ROWS):
                pltpu.sync_copy(rows_v.at[pl.ds(j * 128, 128)],
                                acc_sh.at[tgt_v.at[u * IDX_ROWS + j]],
                                add=True)
            other = base0 + (1 - u) * SLOT
            if u == 1:
                pltpu.sync_copy(acc_sh.at[pl.ds(other, OUT_ROWS)],
                                out_hbm.at[b - 1])
            else:
                @pl.when(t > 0)
                def _():
                    pltpu.sync_copy(acc_sh.at[pl.ds(other, OUT_ROWS)],
                                    out_hbm.at[b - 1])

    pltpu.sync_copy(acc_sh.at[pl.ds(base0 + SLOT, OUT_ROWS)],
                    out_hbm.at[first + BATCHES_PER_W - 1])


@jax.jit
def _atom_feature_sc(x3d, atom_table, tmpl, tgt):
    mesh = plsc.VectorSubcoreMesh(core_axis_name="c", subcore_axis_name="s")
    kfn = pl.kernel(
        _sc_body,
        out_type=jax.ShapeDtypeStruct((B, OUT_ROWS, D), jnp.float32),
        mesh=mesh,
        scratch_types=[
            pltpu.VMEM((IDX_ROWS, 128), jnp.int32),
            pltpu.VMEM((2 * IDX_ROWS, 128), jnp.int32),
            pltpu.VMEM((IDX_PER_BATCH, D), jnp.float32),
            pltpu.VMEM((OUT_ROWS, D), jnp.float32),
            pltpu.VMEM_SHARED((16 * 2 * SLOT, D), jnp.float32),
        ],
    )
    return kfn(x3d, atom_table, tmpl, tgt)


_NODE_ORDER = np.arange(N).reshape(N // 4, 4).T.reshape(-1)


def kernel(x, atom_table, graph_token):
    x3d = x[:, _NODE_ORDER, :].reshape(B, IDX_ROWS, 128).astype(jnp.int32)
    tmpl = jnp.concatenate(
        [graph_token.astype(jnp.float32),
         jnp.zeros((N, D), jnp.float32)], axis=0)
    node = jnp.asarray(_NODE_ORDER, jnp.int32).reshape(IDX_ROWS, 16)
    node = jnp.repeat(node, F, axis=1)
    tgt = (jnp.arange(16, dtype=jnp.int32)[:, None, None, None] * (2 * SLOT)
           + jnp.arange(2, dtype=jnp.int32)[None, :, None, None] * SLOT
           + 1 + node[None, None]).astype(jnp.int32)
    tgt = tgt.reshape(16, 2 * IDX_ROWS, 128)
    return _atom_feature_sc(x3d, atom_table, tmpl, tgt)


# speedup vs baseline: 9.9579x; 1.7840x over previous
import numpy as np
import jax
import jax.numpy as jnp
from jax import lax
from jax.experimental import pallas as pl
from jax.experimental.pallas import tpu as pltpu
from jax.experimental.pallas import tpu_sc as plsc

B, N, F, D = 1024, 64, 8, 128
IDX_PER_BATCH = N * F
IDX_ROWS = IDX_PER_BATCH // 128
NW = 32
BATCHES_PER_W = B // NW
OUT_ROWS = N + 1
SLOT = 72


def _sc_body(x_hbm, tab_hbm, tmpl_hbm, tgt_hbm, out_hbm,
             idx_v, tgt_v, rows_v, tmpl_v, acc_sh,
             g_sems, s_sems, i_sem, o_sem, r_sem):
    c = lax.axis_index("c")
    s = lax.axis_index("s")
    wid = s * 2 + c
    first = wid * BATCHES_PER_W
    base0 = s * 2 * SLOT

    pltpu.sync_copy(tmpl_hbm, tmpl_v)
    pltpu.sync_copy(tgt_hbm.at[s], tgt_v)

    def g_desc(iu, j, buf):
        return pltpu.make_async_copy(
            tab_hbm.at[idx_v.at[iu * IDX_ROWS + j]],
            rows_v.at[pl.ds(buf * 128, 128)], g_sems.at[buf])

    def s_desc(u, j, buf):
        return pltpu.make_async_copy(
            rows_v.at[pl.ds(buf * 128, 128)],
            acc_sh.at[tgt_v.at[u * IDX_ROWS + j]], s_sems.at[buf])

    def i_desc(b, iu):
        return pltpu.make_async_copy(
            x_hbm.at[b], idx_v.at[pl.ds(iu * IDX_ROWS, IDX_ROWS)], i_sem)

    def r_desc(u):
        return pltpu.make_async_copy(
            tmpl_v, acc_sh.at[pl.ds(base0 + u * SLOT, OUT_ROWS)], r_sem)

    def o_desc(b, u):
        return pltpu.make_async_copy(
            acc_sh.at[pl.ds(base0 + u * SLOT, OUT_ROWS)],
            out_hbm.at[b], o_sem)

    def do_batch(b, u, fb=False, sb=False):
        if not fb and not sb:
            o_desc(b - 2, u).wait()
        r_desc(u).start()
        i_desc(jnp.minimum(b + 1, B - 1), 1 - u).start()
        for j in range(IDX_ROWS):
            g_desc(u, j, j).wait()
            if j == 0:
                r_desc(u).wait()
            s_desc(u, j, j).start(add=True)
            nxt = (j + 3) % IDX_ROWS
            if j == 0:
                if not fb:
                    s_desc(1 - u, 3, nxt).wait()
                    g_desc(u, 3, nxt).start()
                else:
                    g_desc(u, 3, nxt).start()
            else:
                s_desc(u, nxt, nxt).wait()
                if j == 1:
                    i_desc(b, 1 - u).wait()
                g_desc(1 - u, j - 1, nxt).start()
        if not fb:
            o_desc(b - 1, 1 - u).start()

    pltpu.sync_copy(x_hbm.at[first], idx_v.at[pl.ds(0, IDX_ROWS)])
    for j in range(IDX_ROWS - 1):
        g_desc(0, j, j).start()

    do_batch(first, 0, fb=True)
    do_batch(first + 1, 1, sb=True)

    @pl.loop(2, BATCHES_PER_W, step=2)
    def _(t):
        do_batch(first + t, 0)
        do_batch(first + t + 1, 1)

    for j in range(IDX_ROWS - 1):
        g_desc(0, j, j).wait()
    s_desc(1, 3, 3).wait()
    o_desc(first + BATCHES_PER_W - 2, 0).wait()
    pltpu.sync_copy(acc_sh.at[pl.ds(base0 + SLOT, OUT_ROWS)],
                    out_hbm.at[first + BATCHES_PER_W - 1])


@jax.jit
def _atom_feature_sc(x3d, atom_table, tmpl, tgt):
    mesh = plsc.VectorSubcoreMesh(core_axis_name="c", subcore_axis_name="s")
    kfn = pl.kernel(
        _sc_body,
        out_type=jax.ShapeDtypeStruct((B, OUT_ROWS, D), jnp.float32),
        mesh=mesh,
        scratch_types=[
            pltpu.VMEM((2 * IDX_ROWS, 128), jnp.int32),
            pltpu.VMEM((2 * IDX_ROWS, 128), jnp.int32),
            pltpu.VMEM((IDX_ROWS * 128, D), jnp.float32),
            pltpu.VMEM((OUT_ROWS, D), jnp.float32),
            pltpu.VMEM_SHARED((16 * 2 * SLOT, D), jnp.float32),
            pltpu.SemaphoreType.DMA((IDX_ROWS,)),
            pltpu.SemaphoreType.DMA((IDX_ROWS,)),
            pltpu.SemaphoreType.DMA,
            pltpu.SemaphoreType.DMA,
            pltpu.SemaphoreType.DMA,
        ],
    )
    return kfn(x3d, atom_table, tmpl, tgt)


_NODE_ORDER = np.arange(N).reshape(N // 4, 4).T.reshape(-1)


def kernel(x, atom_table, graph_token):
    x3d = x[:, _NODE_ORDER, :].reshape(B, IDX_ROWS, 128).astype(jnp.int32)
    tmpl = jnp.concatenate(
        [graph_token.astype(jnp.float32),
         jnp.zeros((N, D), jnp.float32)], axis=0)
    node = jnp.asarray(_NODE_ORDER, jnp.int32).reshape(IDX_ROWS, 16)
    node = jnp.repeat(node, F, axis=1)
    tgt = (jnp.arange(16, dtype=jnp.int32)[:, None, None, None] * (2 * SLOT)
           + jnp.arange(2, dtype=jnp.int32)[None, :, None, None] * SLOT
           + 1 + node[None, None]).astype(jnp.int32)
    tgt = tgt.reshape(16, 2 * IDX_ROWS, 128)
    return _atom_feature_sc(x3d, atom_table, tmpl, tgt)
